# baseline (speedup 1.0000x reference)
import functools

import jax
import jax.numpy as jnp
from jax import lax
from jax.experimental import pallas as pl
from jax.experimental.pallas import tpu as pltpu
from jax.experimental.pallas import tpu_sc as plsc

VOCAB = 1000000
EMBED_DIM = 64
BATCH = 4096
HIST = 200

NUM_SC = 2
NUM_SUBCORES = 16
NUM_WORKERS = NUM_SC * NUM_SUBCORES
ROWS_PER_W = BATCH // NUM_WORKERS
TOK_PER_W = ROWS_PER_W * HIST
LANES = 16
GROUPS = ROWS_PER_W // LANES

ROW_BLOCK = 25000


def _scores_body(table_ref, w_ref, b_ref, out_ref):
    t = table_ref[...]
    w = w_ref[...]
    s = jnp.sum(t * w, axis=1, keepdims=True)
    out_ref[...] = (s + b_ref[0, 0]) * (1.0 / HIST)


def _compute_scores(table, fc_w, fc_b):
    grid = VOCAB // ROW_BLOCK
    return pl.pallas_call(
        _scores_body,
        grid=(grid,),
        in_specs=[
            pl.BlockSpec((ROW_BLOCK, EMBED_DIM), lambda i: (i, 0)),
            pl.BlockSpec((1, EMBED_DIM), lambda i: (0, 0)),
            pl.BlockSpec(memory_space=pltpu.SMEM),
        ],
        out_specs=pl.BlockSpec((ROW_BLOCK, 1), lambda i: (i, 0)),
        out_shape=jax.ShapeDtypeStruct((VOCAB, 1), jnp.float32),
    )(table, fc_w, fc_b.reshape(1, 1))


def _pool_body(xprep_hbm, scores_hbm, out_hbm, idx_v, vals_v, res_v, sem):
    wid = lax.axis_index("c") * NUM_SUBCORES + lax.axis_index("s")
    base = wid * TOK_PER_W

    pltpu.sync_copy(xprep_hbm.at[pl.ds(base, TOK_PER_W)], idx_v)
    pltpu.async_copy(scores_hbm.at[idx_v], vals_v, sem).wait()

    def body(l, accs):
        off = l * ROWS_PER_W
        return tuple(
            accs[g] + vals_v[pl.ds(off + g * LANES, LANES)]
            for g in range(GROUPS)
        )

    init = tuple(jnp.zeros((LANES,), jnp.float32) for _ in range(GROUPS))
    accs = lax.fori_loop(0, HIST, body, init)

    for g in range(GROUPS):
        z = accs[g]
        res_v[pl.ds(g * LANES, LANES)] = 1.0 / (1.0 + jnp.exp(-z))
    pltpu.sync_copy(res_v, out_hbm.at[pl.ds(wid * ROWS_PER_W, ROWS_PER_W)])


@functools.cache
def _make_pool():
    return pl.kernel(
        _pool_body,
        out_type=jax.ShapeDtypeStruct((BATCH,), jnp.float32),
        mesh=plsc.VectorSubcoreMesh(
            core_axis_name="c", subcore_axis_name="s",
            num_cores=NUM_SC, num_subcores=NUM_SUBCORES),
        scratch_types=[
            pltpu.VMEM((TOK_PER_W,), jnp.int32),
            pltpu.VMEM((TOK_PER_W,), jnp.float32),
            pltpu.VMEM((ROWS_PER_W,), jnp.float32),
            pltpu.SemaphoreType.DMA,
        ],
    )


def kernel(x, table, fc_w, fc_b):
    scores = _compute_scores(table, fc_w, fc_b)
    xprep = (
        x.astype(jnp.int32)
        .reshape(NUM_WORKERS, ROWS_PER_W, HIST)
        .transpose(0, 2, 1)
        .reshape(-1)
    )
    out = _make_pool()(xprep, scores.reshape(-1))
    return out.reshape(BATCH, 1)
